# Initial kernel scaffold; baseline (speedup 1.0000x reference)
import jax
import jax.numpy as jnp
from jax.experimental import pallas as pl


def kernel(x, embed, W1, b1, W2, b2, ln_g, ln_b, Wk, Wv, Wq, Wr, alpha, Wout, bout):
    raise NotImplementedError("write your pallas kernel here")


# speedup vs baseline: 3.1842x; 3.1842x over previous
import functools

import jax
import jax.numpy as jnp
from jax.experimental import pallas as pl
from jax.experimental.pallas import tpu as pltpu

H = 64
VOCAB = 64
NUM_PAIRS = 6
ENERGY_THRESHOLD = 0.4
LN_EPS = 1e-5
NEG_BIG = -1e30


def _encoder_kernel(x_ref, embedt_ref, w1_ref, b1f_ref, w2_ref, b2f_ref,
                    gf_ref, blnf_ref, out_ref):
    x = x_ref[...]
    lb, bb = x.shape
    embedt = embedt_ref[...]
    w1 = w1_ref[...]
    w2 = w2_ref[...]
    b1f = b1f_ref[...]
    b2f = b2f_ref[...]
    gf = gf_ref[...]
    blnf = blnf_ref[...]
    iota_v = jax.lax.broadcasted_iota(jnp.int32, (VOCAB, bb), 0)
    outs = []
    for l in range(lb):
        oh = (iota_v == x[l:l + 1, :]).astype(jnp.float32)
        e = jnp.dot(embedt, oh, preferred_element_type=jnp.float32, precision=jax.lax.Precision.HIGHEST)
        z1 = jnp.maximum(
            jnp.dot(w1.astype(jnp.bfloat16), e.astype(jnp.bfloat16),
                    preferred_element_type=jnp.float32) + b1f, 0.0)
        z2 = jnp.dot(w2.astype(jnp.bfloat16), z1.astype(jnp.bfloat16),
                     preferred_element_type=jnp.float32) + b2f
        y = e + z2
        mu = jnp.mean(y, axis=0, keepdims=True)
        d = y - mu
        var = jnp.mean(d * d, axis=0, keepdims=True)
        o = d * jax.lax.rsqrt(var + LN_EPS) * gf + blnf
        outs.append(jnp.swapaxes(o, 0, 1)[None])
    out_ref[...] = jnp.concatenate(outs, axis=0)


def _scan_kernel(hT_ref, wkt_ref, wvt_ref, wqt_ref, wrt_ref, wot_ref,
                 bout_ref, alpha_ref, out_ref, m_ref, k_ref, v_ref):
    t = pl.program_id(1)
    n_t = pl.num_programs(1)
    c_blk, b_blk, _ = hT_ref.shape

    @pl.when(t == 0)
    def _():
        m_ref[...] = jnp.zeros_like(m_ref)

    h_c = hT_ref[...]
    flat = h_c.reshape(c_blk * b_blk, H)
    flat16 = flat.astype(jnp.bfloat16)
    k_all = jnp.dot(flat16, wkt_ref[...].astype(jnp.bfloat16),
                    preferred_element_type=jnp.float32)
    k_nrm = jnp.sqrt(jnp.sum(k_all * k_all, axis=-1, keepdims=True))
    k_all = k_all / jnp.maximum(k_nrm, 1e-12)
    v_all = jnp.dot(flat16, wvt_ref[...].astype(jnp.bfloat16),
                    preferred_element_type=jnp.float32)
    k_ref[...] = k_all.reshape(c_blk, b_blk, H)
    v_ref[...] = v_all.reshape(c_blk, b_blk, H)

    thr = jnp.float32(ENERGY_THRESHOLD)

    def body(c, carry):
        k = k_ref[c]
        v = v_ref[c]
        m = m_ref[...]
        pred = jnp.sum(m * k[:, None, :], axis=-1)
        delta = v - pred
        dn = jnp.sqrt(jnp.sum(delta * delta, axis=-1, keepdims=True))
        vn = jnp.sqrt(jnp.sum(v * v, axis=-1, keepdims=True))
        gate = jnp.where(dn > thr * vn, 1.0, 0.0)
        gd = gate * delta
        m_ref[...] = m + gd[:, :, None] * k[:, None, :]
        return carry

    jax.lax.fori_loop(0, c_blk, body, 0)

    @pl.when(t == n_t - 1)
    def _():
        m = m_ref[...]
        h_last = h_c[c_blk - 1]
        q = jnp.dot(h_last.astype(jnp.bfloat16),
                    wqt_ref[...].astype(jnp.bfloat16),
                    preferred_element_type=jnp.float32)
        qr = jnp.dot(q.astype(jnp.bfloat16), wrt_ref[...].astype(jnp.bfloat16),
                     preferred_element_type=jnp.float32)
        n2 = jnp.sum(m * m, axis=1)
        k_s = min(NUM_PAIRS + 2, H)
        lane = jax.lax.broadcasted_iota(jnp.int32, n2.shape, 1)
        cur = n2
        sel = jnp.zeros(n2.shape, jnp.bool_)
        for _ in range(k_s):
            idx = jnp.argmax(cur, axis=-1)
            pick = lane == idx[:, None]
            sel = jnp.logical_or(sel, pick)
            cur = jnp.where(pick, -1.0, cur)
        lg = jnp.sum(m * qr[:, :, None], axis=1)
        lg = lg * jnp.float32(1.0 / (H ** 0.5))
        lg = jnp.where(sel, lg, NEG_BIG)
        mx = jnp.max(lg, axis=-1, keepdims=True)
        ex = jnp.exp(lg - mx)
        attn = ex / jnp.sum(ex, axis=-1, keepdims=True)
        retro = jnp.sum(m * attn[:, None, :], axis=-1)
        mctx = jnp.sum(m * q[:, None, :], axis=-1)
        a = jax.nn.sigmoid(alpha_ref[0, 0])
        mixed = jnp.maximum(a * retro + (1.0 - a) * mctx, 0.0)
        out_ref[...] = (jnp.dot(mixed.astype(jnp.bfloat16),
                                wot_ref[...].astype(jnp.bfloat16),
                                preferred_element_type=jnp.float32)
                        + bout_ref[...])


@functools.partial(jax.jit, static_argnames=())
def kernel(x, embed, W1, b1, W2, b2, ln_g, ln_b, Wk, Wv, Wq, Wr, alpha,
           Wout, bout):
    B, L = x.shape
    n_b = 2
    b_blk = B // n_b
    l_blk = 8
    c_blk = 128 if L % 128 == 0 else L

    xT = jnp.swapaxes(x, 0, 1).astype(jnp.int32)
    row = lambda a: a.reshape(1, -1)
    fmaj = lambda a: jnp.tile(a.reshape(-1, 1), (1, b_blk))

    hT = pl.pallas_call(
        _encoder_kernel,
        out_shape=jax.ShapeDtypeStruct((L, B, H), jnp.float32),
        grid=(n_b, L // l_blk),
        in_specs=[
            pl.BlockSpec((l_blk, b_blk), lambda b, l: (l, b)),
            pl.BlockSpec((H, VOCAB), lambda b, l: (0, 0)),
            pl.BlockSpec((2 * H, H), lambda b, l: (0, 0)),
            pl.BlockSpec((2 * H, b_blk), lambda b, l: (0, 0)),
            pl.BlockSpec((H, 2 * H), lambda b, l: (0, 0)),
            pl.BlockSpec((H, b_blk), lambda b, l: (0, 0)),
            pl.BlockSpec((H, b_blk), lambda b, l: (0, 0)),
            pl.BlockSpec((H, b_blk), lambda b, l: (0, 0)),
        ],
        out_specs=pl.BlockSpec((l_blk, b_blk, H), lambda b, l: (l, b, 0)),
        compiler_params=pltpu.CompilerParams(
            dimension_semantics=("parallel", "arbitrary"),
        ),
        name="delta_encoder",
    )(xT, embed.T, W1, fmaj(b1), W2, fmaj(b2), fmaj(ln_g), fmaj(ln_b))

    out = pl.pallas_call(
        _scan_kernel,
        out_shape=jax.ShapeDtypeStruct((B, VOCAB), jnp.float32),
        grid=(n_b, L // c_blk),
        in_specs=[
            pl.BlockSpec((c_blk, b_blk, H), lambda b, t: (t, b, 0)),
            pl.BlockSpec((H, H), lambda b, t: (0, 0)),
            pl.BlockSpec((H, H), lambda b, t: (0, 0)),
            pl.BlockSpec((H, H), lambda b, t: (0, 0)),
            pl.BlockSpec((H, H), lambda b, t: (0, 0)),
            pl.BlockSpec((H, VOCAB), lambda b, t: (0, 0)),
            pl.BlockSpec((1, VOCAB), lambda b, t: (0, 0)),
            pl.BlockSpec((1, 1), lambda b, t: (0, 0)),
        ],
        out_specs=pl.BlockSpec((b_blk, VOCAB), lambda b, t: (b, 0)),
        scratch_shapes=[
            pltpu.VMEM((b_blk, H, H), jnp.float32),
            pltpu.VMEM((c_blk, b_blk, H), jnp.float32),
            pltpu.VMEM((c_blk, b_blk, H), jnp.float32),
        ],
        compiler_params=pltpu.CompilerParams(
            dimension_semantics=("parallel", "arbitrary"),
        ),
        name="delta_scan",
    )(hT, Wk.T, Wv.T, Wq.T, Wr.T, Wout.T,
      row(bout), jnp.reshape(alpha, (1, 1)).astype(jnp.float32))

    return out
